# Initial kernel scaffold; baseline (speedup 1.0000x reference)
import jax
import jax.numpy as jnp
from jax.experimental import pallas as pl


def kernel(user_table, poi_table, cat_table, hour_table, day_table, qk_table, user_idx, poi_idx, category_idx, hour_idx, day_idx, quadkey_idx):
    raise NotImplementedError("write your pallas kernel here")


# speedup vs baseline: 1.8537x; 1.8537x over previous
import functools

import jax
import jax.numpy as jnp
from jax import lax
from jax.experimental import pallas as pl
from jax.experimental.pallas import tpu as pltpu
from jax.experimental.pallas import tpu_sc as plsc

D = 32
NTAB = 6


def _build_sc_kernel(n_total: int, n_per_w: int, chunk: int, num_cores: int):
    n_ch = n_per_w // chunk
    mesh = plsc.VectorSubcoreMesh(core_axis_name="c", subcore_axis_name="s")

    @functools.partial(
        pl.kernel,
        mesh=mesh,
        out_type=jax.ShapeDtypeStruct((n_total, NTAB * D), jnp.float32),
        compiler_params=pltpu.CompilerParams(use_tc_tiling_on_sc=False),
        scratch_types=[
            pltpu.VMEM((chunk,), jnp.int32),
            pltpu.VMEM((chunk, D), jnp.float32),
            pltpu.SemaphoreType.DMA,
        ],
    )
    def sc_kernel(u_t, p_t, c_t, h_t, d_t, q_t,
                  u_i, p_i, c_i, h_i, d_i, q_i,
                  out, idx_v, rows_v, sem):
        wid = lax.axis_index("s") * num_cores + lax.axis_index("c")
        base = wid * n_per_w
        pairs = [(u_t, u_i), (p_t, p_i), (c_t, c_i),
                 (h_t, h_i), (d_t, d_i), (q_t, q_i)]
        for t, (tab, idx) in enumerate(pairs):
            for ci in range(n_ch):
                off = base + ci * chunk
                pltpu.sync_copy(idx.at[pl.ds(off, chunk)], idx_v)
                pltpu.async_copy(tab.at[idx_v], rows_v, sem).wait()
                pltpu.sync_copy(rows_v, out.at[pl.ds(off, chunk), pl.ds(t * D, D)])

    return sc_kernel


def kernel(user_table, poi_table, cat_table, hour_table, day_table, qk_table,
           user_idx, poi_idx, category_idx, hour_idx, day_idx, quadkey_idx):
    B, L = user_idx.shape
    n_total = B * L
    info = plsc.get_sparse_core_info()
    num_workers = info.num_cores * info.num_subcores
    n_per_w = n_total // num_workers
    chunk = 3200
    assert n_per_w % chunk == 0

    sc = _build_sc_kernel(n_total, n_per_w, chunk, info.num_cores)
    idxs = [x.reshape(-1) for x in (user_idx, poi_idx, category_idx,
                                    hour_idx, day_idx, quadkey_idx)]
    out = sc(user_table, poi_table, cat_table, hour_table, day_table, qk_table,
             *idxs)
    return out.reshape(B, L, NTAB * D)


# speedup vs baseline: 1.9544x; 1.0544x over previous
import functools

import jax
import jax.numpy as jnp
from jax import lax
from jax.experimental import pallas as pl
from jax.experimental.pallas import tpu as pltpu
from jax.experimental.pallas import tpu_sc as plsc

D = 32
NTAB = 6
CHUNK = 200


def _build_sc_kernel(n_total: int, n_per_w: int, num_cores: int):
    n_ch = n_per_w // CHUNK
    assert n_per_w % CHUNK == 0 and n_ch % 2 == 0
    mesh = plsc.VectorSubcoreMesh(core_axis_name="c", subcore_axis_name="s")

    @functools.partial(
        pl.kernel,
        mesh=mesh,
        out_type=jax.ShapeDtypeStruct((n_total, NTAB * D), jnp.float32),
        compiler_params=pltpu.CompilerParams(use_tc_tiling_on_sc=False),
        scratch_types=[
            pltpu.VMEM((NTAB, n_per_w), jnp.int32),
            pltpu.VMEM((2, NTAB, CHUNK, D), jnp.float32),
            pltpu.SemaphoreType.DMA,
            pltpu.SemaphoreType.DMA,
            pltpu.SemaphoreType.DMA,
            pltpu.SemaphoreType.DMA,
        ],
    )
    def sc_kernel(u_t, p_t, c_t, h_t, d_t, q_t,
                  u_i, p_i, c_i, h_i, d_i, q_i,
                  out, idx_all, rows, sg0, sg1, sw0, sw1):
        wid = lax.axis_index("s") * num_cores + lax.axis_index("c")
        base = wid * n_per_w
        tabs = [u_t, p_t, c_t, h_t, d_t, q_t]
        idxs = [u_i, p_i, c_i, h_i, d_i, q_i]
        sg = [sg0, sg1]
        sw = [sw0, sw1]

        for t in range(NTAB):
            pltpu.sync_copy(idxs[t].at[pl.ds(base, n_per_w)], idx_all.at[t])

        def gathers(ci, b):
            off = ci * CHUNK
            for t in range(NTAB):
                pltpu.async_copy(
                    tabs[t].at[idx_all.at[t, pl.ds(off, CHUNK)]],
                    rows.at[b, t], sg[b])

        def wait_g(b):
            for t in range(NTAB):
                pltpu.make_async_copy(
                    out.at[pl.ds(0, CHUNK), pl.ds(0, D)],
                    rows.at[b, t], sg[b]).wait()

        def writes(ci, b):
            off = base + ci * CHUNK
            for t in range(NTAB):
                pltpu.async_copy(
                    rows.at[b, t],
                    out.at[pl.ds(off, CHUNK), pl.ds(t * D, D)], sw[b])

        def wait_w(b):
            for t in range(NTAB):
                pltpu.make_async_copy(
                    rows.at[b, t],
                    out.at[pl.ds(0, CHUNK), pl.ds(0, D)], sw[b]).wait()

        gathers(0, 0)

        def body(i, carry):
            for k in range(2):
                ci = i * 2 + k
                b = k
                wait_g(b)
                writes(ci, b)

                @pl.when(ci + 1 < n_ch)
                def _issue_next():
                    @pl.when(ci > 0)
                    def _drain_writes():
                        wait_w(1 - b)
                    gathers(ci + 1, 1 - b)
            return carry

        lax.fori_loop(0, n_ch // 2, body, 0)
        wait_w(0)
        wait_w(1)

    return sc_kernel


def kernel(user_table, poi_table, cat_table, hour_table, day_table, qk_table,
           user_idx, poi_idx, category_idx, hour_idx, day_idx, quadkey_idx):
    B, L = user_idx.shape
    n_total = B * L
    info = plsc.get_sparse_core_info()
    num_workers = info.num_cores * info.num_subcores
    n_per_w = n_total // num_workers

    sc = _build_sc_kernel(n_total, n_per_w, info.num_cores)
    idxs = [x.reshape(-1) for x in (user_idx, poi_idx, category_idx,
                                    hour_idx, day_idx, quadkey_idx)]
    out = sc(user_table, poi_table, cat_table, hour_table, day_table, qk_table,
             *idxs)
    return out.reshape(B, L, NTAB * D)
